# Initial kernel scaffold; baseline (speedup 1.0000x reference)
import jax
import jax.numpy as jnp
from jax import lax
from jax.experimental import pallas as pl
from jax.experimental.pallas import tpu as pltpu
from jax.experimental.pallas import tpu_sc as plsc

NUM_EMBEDDINGS = 1000000
EMBEDDING_DIM = 32
BATCH = 16384
FIELDS = 26

_B = BATCH * FIELDS
_NW = 32
_PER_W = _B // _NW
_CHUNK = 1024
_NCHUNK = _PER_W // _CHUNK


def _body(table_hbm, idx_hbm, out_hbm, idx_v, rows_v, gsem, ssem):
    nc = 2
    wid = lax.axis_index("s") * nc + lax.axis_index("c")
    base = wid * _PER_W

    pltpu.sync_copy(idx_hbm.at[pl.ds(base, _PER_W)], idx_v)

    pltpu.async_copy(
        table_hbm.at[idx_v.at[pl.ds(0, _CHUNK)]], rows_v.at[0], gsem)
    for c in range(_NCHUNK):
        buf = c % 2
        if c + 1 < _NCHUNK:
            pltpu.async_copy(
                table_hbm.at[idx_v.at[pl.ds((c + 1) * _CHUNK, _CHUNK)]],
                rows_v.at[1 - buf], gsem)
        pltpu.make_async_copy(
            table_hbm.at[idx_v.at[pl.ds(c * _CHUNK, _CHUNK)]],
            rows_v.at[buf], gsem).wait()
        if c >= 2:
            pltpu.make_async_copy(
                rows_v.at[buf],
                out_hbm.at[pl.ds(base + (c - 2) * _CHUNK, _CHUNK)],
                ssem).wait()
        pltpu.async_copy(
            rows_v.at[buf],
            out_hbm.at[pl.ds(base + c * _CHUNK, _CHUNK)], ssem)
    for c in (_NCHUNK - 2, _NCHUNK - 1):
        pltpu.make_async_copy(
            rows_v.at[c % 2],
            out_hbm.at[pl.ds(base + c * _CHUNK, _CHUNK)], ssem).wait()


@jax.jit
def _embed(idx_flat, weight):
    mesh = plsc.VectorSubcoreMesh(core_axis_name="c", subcore_axis_name="s")
    fn = pl.kernel(
        _body,
        out_type=jax.ShapeDtypeStruct((_B, EMBEDDING_DIM), jnp.float32),
        mesh=mesh,
        scratch_types=[
            pltpu.VMEM((_PER_W,), jnp.int32),
            pltpu.VMEM((2, _CHUNK, EMBEDDING_DIM), jnp.float32),
            pltpu.SemaphoreType.DMA,
            pltpu.SemaphoreType.DMA,
        ],
    )
    return fn(weight, idx_flat)


def kernel(input, weight):
    out = _embed(input.reshape(-1), weight)
    return out.reshape(BATCH, FIELDS, EMBEDDING_DIM)


# speedup vs baseline: 1.5746x; 1.5746x over previous
import jax
import jax.numpy as jnp
from jax import lax
from jax.experimental import pallas as pl
from jax.experimental.pallas import tpu as pltpu
from jax.experimental.pallas import tpu_sc as plsc

NUM_EMBEDDINGS = 1000000
EMBEDDING_DIM = 32
BATCH = 16384
FIELDS = 26

_B = BATCH * FIELDS
_NW = 32
_PER_W = _B // _NW
_CHUNK = 1024
_NCHUNK = _PER_W // _CHUNK


def _body(table_hbm, idx_hbm, out_hbm, idx_v, rows_v,
          gsem0, gsem1, ssem0, ssem1):
    nc = 2
    wid = lax.axis_index("s") * nc + lax.axis_index("c")
    base = wid * _PER_W
    gsem = (gsem0, gsem1)
    ssem = (ssem0, ssem1)

    def gather(c, buf):
        return pltpu.async_copy(
            table_hbm.at[idx_v.at[pl.ds(c * _CHUNK, _CHUNK)]],
            rows_v.at[buf], gsem[buf])

    def store(c, buf):
        return pltpu.async_copy(
            rows_v.at[buf],
            out_hbm.at[pl.ds(base + c * _CHUNK, _CHUNK)], ssem[buf])

    pltpu.sync_copy(idx_hbm.at[pl.ds(base, _PER_W)], idx_v)

    pending_g = gather(0, 0)
    pending_s = [None, None]
    for c in range(_NCHUNK):
        buf = c % 2
        nbuf = 1 - buf
        next_g = None
        if c + 1 < _NCHUNK:
            if pending_s[nbuf] is not None:
                pending_s[nbuf].wait()
                pending_s[nbuf] = None
            next_g = gather(c + 1, nbuf)
        pending_g.wait()
        pending_g = next_g
        pending_s[buf] = store(c, buf)
    for s in pending_s:
        if s is not None:
            s.wait()


@jax.jit
def _embed(idx_flat, weight):
    mesh = plsc.VectorSubcoreMesh(core_axis_name="c", subcore_axis_name="s")
    fn = pl.kernel(
        _body,
        out_type=jax.ShapeDtypeStruct((_B, EMBEDDING_DIM), jnp.float32),
        mesh=mesh,
        scratch_types=[
            pltpu.VMEM((_PER_W,), jnp.int32),
            pltpu.VMEM((2, _CHUNK, EMBEDDING_DIM), jnp.float32),
            pltpu.SemaphoreType.DMA,
            pltpu.SemaphoreType.DMA,
            pltpu.SemaphoreType.DMA,
            pltpu.SemaphoreType.DMA,
        ],
        compiler_params=pltpu.CompilerParams(use_tc_tiling_on_sc=False),
    )
    return fn(weight, idx_flat)


def kernel(input, weight):
    out = _embed(input.reshape(-1), weight)
    return out.reshape(BATCH, FIELDS, EMBEDDING_DIM)
